# baseline (speedup 1.0000x reference)
import jax
import jax.numpy as jnp
from jax import lax
from jax.experimental import pallas as pl
from jax.experimental.pallas import tpu as pltpu
from jax.experimental.pallas import tpu_sc as plsc

B = 16384
NUM_CORES = 2
NUM_SUBCORES = 16
NUM_WORKERS = NUM_CORES * NUM_SUBCORES
BPW = B // NUM_WORKERS
CHUNK = 128
NCHUNK = BPW // CHUNK
D = 32
INV_B = 1.0 / B

_mesh = plsc.VectorSubcoreMesh(core_axis_name="c", subcore_axis_name="s")


def _sc_body(users_ref, articles_ref, ratings_ref, utab_ref, atab_ref, wb_ref,
             out_ref, lp_ref,
             uidx_v, aidx_v, urows_v, arows_v, rat_v, out_v, wb_v, part_v,
             all_v, shared_v, sem):
    c = lax.axis_index("c")
    s = lax.axis_index("s")
    wid = s * NUM_CORES + c

    pltpu.sync_copy(users_ref.at[wid], uidx_v)
    pltpu.sync_copy(articles_ref.at[wid], aidx_v)
    pltpu.sync_copy(ratings_ref.at[wid], rat_v)
    pltpu.sync_copy(wb_ref, wb_v)

    cps = []
    for j in range(NCHUNK):
        cps.append(pltpu.async_copy(
            utab_ref.at[uidx_v.at[j]], urows_v.at[pl.ds(j * CHUNK, CHUNK)], sem))
        cps.append(pltpu.async_copy(
            atab_ref.at[aidx_v.at[j]], arows_v.at[pl.ds(j * CHUNK, CHUNK)], sem))
    for cp in cps:
        cp.wait()

    wvecs = [wb_v[pl.ds(k * 16, 16)] for k in range(4)]
    bias = wb_v[pl.ds(64, 16)][0]
    lane = lax.iota(jnp.int32, 16)

    def block(b, lacc):
        base = pl.multiple_of(b * 16, 16)
        rows16 = base + lane
        acc = jnp.full((16,), bias, jnp.float32)
        for d in range(D):
            col = plsc.load_gather(urows_v, [rows16, jnp.full((16,), d, jnp.int32)])
            acc = acc + col * wvecs[d // 16][d % 16]
        for d in range(D):
            col = plsc.load_gather(arows_v, [rows16, jnp.full((16,), d, jnp.int32)])
            acc = acc + col * wvecs[(D + d) // 16][d % 16]
        out_v[pl.ds(base, 16)] = acc
        diff = acc - rat_v[pl.ds(base, 16)]
        return lacc + diff * diff

    lacc = lax.fori_loop(0, BPW // 16, block, jnp.zeros((16,), jnp.float32))

    pltpu.sync_copy(out_v, out_ref.at[wid])

    part_v[...] = lacc * INV_B
    pltpu.sync_copy(part_v, shared_v.at[s])
    plsc.subcore_barrier()

    @pl.when(s == 0)
    def _():
        pltpu.sync_copy(shared_v, all_v)
        acc = all_v[0]
        for i in range(1, NUM_SUBCORES):
            acc = acc + all_v[i]
        total = jnp.sum(acc)
        part_v[...] = jnp.where(lane == 0, jnp.full((16,), total, jnp.float32),
                                jnp.zeros((16,), jnp.float32))
        pltpu.sync_copy(part_v, lp_ref.at[c])


_sc_call = pl.kernel(
    _sc_body,
    out_type=(
        jax.ShapeDtypeStruct((NUM_WORKERS, BPW), jnp.float32),
        jax.ShapeDtypeStruct((NUM_CORES, 16), jnp.float32),
    ),
    mesh=_mesh,
    compiler_params=pltpu.CompilerParams(needs_layout_passes=False,
                                         use_tc_tiling_on_sc=False),
    scratch_types=[
        pltpu.VMEM((NCHUNK, CHUNK), jnp.int32),
        pltpu.VMEM((NCHUNK, CHUNK), jnp.int32),
        pltpu.VMEM((BPW, D), jnp.float32),
        pltpu.VMEM((BPW, D), jnp.float32),
        pltpu.VMEM((BPW,), jnp.float32),
        pltpu.VMEM((BPW,), jnp.float32),
        pltpu.VMEM((80,), jnp.float32),
        pltpu.VMEM((16,), jnp.float32),
        pltpu.VMEM((16, 16), jnp.float32),
        pltpu.VMEM_SHARED((16, 16), jnp.float32),
        pltpu.SemaphoreType.DMA,
    ],
)


def kernel(users, articles, ratings, user_table, article_table, fc_w, fc_b):
    users_r = users.astype(jnp.int32).reshape(NUM_WORKERS, NCHUNK, CHUNK)
    articles_r = articles.astype(jnp.int32).reshape(NUM_WORKERS, NCHUNK, CHUNK)
    ratings_r = ratings.reshape(NUM_WORKERS, BPW)
    wb = jnp.concatenate([fc_w.reshape(-1), fc_b.reshape(-1),
                          jnp.zeros(80 - 2 * D - 1, jnp.float32)])
    out_r, lp = _sc_call(users_r, articles_r, ratings_r, user_table,
                         article_table, wb)
    output = out_r.reshape(B, 1)
    loss = lp[0, 0] + lp[1, 0]
    return (output, loss)
